# baseline (speedup 1.0000x reference)
import functools

import jax
import jax.numpy as jnp
from jax import lax
from jax.experimental import pallas as pl
from jax.experimental.pallas import tpu as pltpu
from jax.experimental.pallas import tpu_sc as plsc

_NC = 2
_NS = 16
_NW = _NC * _NS
_B = 128
_NBUF = 2
_KC = 8
_BD = 128
_KSPLIT = (75, 82)
_MESH = dict(core_axis_name="c", subcore_axis_name="s")


def _sc_scatter(g, src3, dst3, zfeat, npad, d, k0, k1):
    rpt = npad // _NS
    kmax = max(k0, k1)
    mesh = plsc.VectorSubcoreMesh(**_MESH)

    @functools.partial(
        pl.kernel,
        out_type=jax.ShapeDtypeStruct((_NC, npad, d), jnp.float32),
        mesh=mesh,
        scratch_types=[
            pltpu.VMEM((kmax, _B), jnp.int32),
            pltpu.VMEM((kmax, _B), jnp.int32),
            pltpu.VMEM((_B, d), jnp.float32),
            pltpu.VMEM_SHARED((npad, d), jnp.float32),
            pltpu.SemaphoreType.DMA,
        ],
    )
    def scat_kernel(g_hbm, src_hbm, dst_hbm, z_hbm, out_hbm,
                    idx_s, idx_d, rows, acc, sem):
        cid = lax.axis_index("c")
        sid = lax.axis_index("s")
        wid = sid * _NC + cid
        sl = pl.ds(sid * rpt, rpt)

        @pl.when(cid == 0)
        def _():
            pltpu.sync_copy(g_hbm.at[sl], acc.at[sl])

        @pl.when(cid != 0)
        def _():
            pltpu.sync_copy(z_hbm.at[sl], acc.at[sl])

        pltpu.sync_copy(src_hbm.at[wid], idx_s)
        pltpu.sync_copy(dst_hbm.at[wid], idx_d)
        plsc.subcore_barrier()

        kw = jnp.where(cid == 0, k0, k1)

        @pl.loop(0, kw)
        def _(j):
            pltpu.async_copy(g_hbm.at[idx_s.at[j]], rows, sem).wait()
            pltpu.sync_copy(rows, acc.at[idx_d.at[j]], add=True)

        plsc.subcore_barrier()
        pltpu.sync_copy(acc.at[sl], out_hbm.at[cid, sl])

    return scat_kernel(g, src3, dst3, zfeat)


def _sc_degree(dst3, ones_feat, zfeat, npad, d, kd):
    rpt = npad // _NS
    nsem = 4
    mesh = plsc.VectorSubcoreMesh(**_MESH)

    @functools.partial(
        pl.kernel,
        out_type=jax.ShapeDtypeStruct((_NC, npad, d), jnp.float32),
        mesh=mesh,
        scratch_types=[
            pltpu.VMEM((kd, _BD), jnp.int32),
            pltpu.VMEM((_BD, d), jnp.float32),
            pltpu.VMEM_SHARED((npad, d), jnp.float32),
        ] + [pltpu.SemaphoreType.DMA] * nsem,
    )
    def deg_kernel(dst_hbm, ones_hbm, z_hbm, out_hbm, idx_d, ones_v, acc,
                   *sems):
        cid = lax.axis_index("c")
        sid = lax.axis_index("s")
        wid = sid * _NC + cid
        sl = pl.ds(sid * rpt, rpt)

        @pl.when(cid == 0)
        def _():
            pltpu.sync_copy(ones_hbm.at[sl], acc.at[sl])

        @pl.when(cid != 0)
        def _():
            pltpu.sync_copy(z_hbm.at[sl], acc.at[sl])

        pltpu.sync_copy(ones_hbm.at[pl.ds(0, _BD)], ones_v)
        pltpu.sync_copy(dst_hbm.at[wid], idx_d)
        plsc.subcore_barrier()

        @pl.loop(0, kd)
        def _(j):
            for s in range(nsem):
                @pl.when(j % nsem == s)
                def _():
                    pltpu.async_copy(ones_v, acc.at[idx_d.at[j]], sems[s],
                                     add=True)

            @pl.when(j >= nsem)
            def _():
                for s in range(nsem):
                    @pl.when(j % nsem == s)
                    def _():
                        pltpu.make_async_copy(
                            ones_v, acc.at[idx_d.at[j - nsem]],
                            sems[s]).wait()

        @pl.loop(kd - nsem, kd)
        def _(j):
            for s in range(nsem):
                @pl.when(j % nsem == s)
                def _():
                    pltpu.make_async_copy(ones_v, acc.at[idx_d.at[j]],
                                          sems[s]).wait()

        plsc.subcore_barrier()
        pltpu.sync_copy(acc.at[sl], out_hbm.at[cid, sl])

    return deg_kernel(dst3, ones_feat, zfeat)


def _dinv_of(degs_blk):
    deg = degs_blk[0, :, 0] + degs_blk[1, :, 0]
    return lax.rsqrt(deg)


def _tc_g1(xpad, W1, degs, npad, d, bm):
    def body(x_ref, w_ref, degs_ref, g_ref, dinv_ref):
        h = jnp.dot(x_ref[...], w_ref[...], preferred_element_type=jnp.float32)
        dinv = _dinv_of(degs_ref)[:, None]
        g_ref[...] = h * dinv
        dinv_ref[...] = jnp.broadcast_to(dinv, (dinv.shape[0], 8))

    return pl.pallas_call(
        body,
        grid=(npad // bm,),
        in_specs=[
            pl.BlockSpec((bm, d), lambda i: (i, 0)),
            pl.BlockSpec((d, d), lambda i: (0, 0)),
            pl.BlockSpec((_NC, bm, d), lambda i: (0, i, 0)),
        ],
        out_specs=[
            pl.BlockSpec((bm, d), lambda i: (i, 0)),
            pl.BlockSpec((bm, 8), lambda i: (i, 0)),
        ],
        out_shape=[
            jax.ShapeDtypeStruct((npad, d), jnp.float32),
            jax.ShapeDtypeStruct((npad, 8), jnp.float32),
        ],
    )(xpad, W1, degs)


def _tc_mid(S1, dinvs, b1, W2, npad, d, bm):
    def body(s_ref, dinv_ref, b_ref, w_ref, pen_ref, g2_ref):
        dinv = dinv_ref[:, 0][:, None]
        ssum = s_ref[0] + s_ref[1]
        pen = jnp.maximum(ssum * dinv + b_ref[...], 0.0)
        pen_ref[...] = pen
        h2 = jnp.dot(pen, w_ref[...], preferred_element_type=jnp.float32)
        g2_ref[...] = h2 * dinv

    return pl.pallas_call(
        body,
        grid=(npad // bm,),
        in_specs=[
            pl.BlockSpec((_NC, bm, d), lambda i: (0, i, 0)),
            pl.BlockSpec((bm, 8), lambda i: (i, 0)),
            pl.BlockSpec((1, d), lambda i: (0, 0)),
            pl.BlockSpec((d, d), lambda i: (0, 0)),
        ],
        out_specs=[
            pl.BlockSpec((bm, d), lambda i: (i, 0)),
            pl.BlockSpec((bm, d), lambda i: (i, 0)),
        ],
        out_shape=[
            jax.ShapeDtypeStruct((npad, d), jnp.float32),
            jax.ShapeDtypeStruct((npad, d), jnp.float32),
        ],
    )(S1, dinvs, b1, W2)


def _tc_out(S2, dinvs, b2, npad, d, bm):
    def body(s_ref, dinv_ref, b_ref, out_ref):
        dinv = dinv_ref[:, 0][:, None]
        out_ref[...] = (s_ref[0] + s_ref[1]) * dinv + b_ref[...]

    return pl.pallas_call(
        body,
        grid=(npad // bm,),
        in_specs=[
            pl.BlockSpec((_NC, bm, d), lambda i: (0, i, 0)),
            pl.BlockSpec((bm, 8), lambda i: (i, 0)),
            pl.BlockSpec((1, d), lambda i: (0, 0)),
        ],
        out_specs=pl.BlockSpec((bm, d), lambda i: (i, 0)),
        out_shape=jax.ShapeDtypeStruct((npad, d), jnp.float32),
    )(S2, dinvs, b2)


def _pad_edges(idx, padval, nw, k, b):
    epad = nw * k * b
    pad = jnp.full((epad - idx.shape[0],), padval, jnp.int32)
    return jnp.concatenate([idx, pad]).reshape(nw, k, b)


def _split_edges(idx, padval, k0, k1, b):
    na = _NS * k0 * b
    nb = _NS * k1 * b
    pad = jnp.full((na + nb - idx.shape[0],), padval, jnp.int32)
    full = jnp.concatenate([idx, pad])
    part_a = full[:na].reshape(_NS, k0, b)
    part_b = full[na:].reshape(_NS, k1, b)
    kmax = max(k0, k1)

    def widen(p, kk):
        if kk == kmax:
            return p
        fill = jnp.full((_NS, kmax - kk, b), padval, jnp.int32)
        return jnp.concatenate([p, fill], axis=1)

    return jnp.stack([widen(part_a, k0), widen(part_b, k1)],
                     axis=1).reshape(_NW, kmax, b)


def kernel(x, edge_index, W1, b1, W2, b2):
    n, d = x.shape
    e = edge_index.shape[1]
    bm = 256
    npad = -(-n // bm) * bm
    k0, k1 = _KSPLIT
    kd = -(-(-(-e // _NW)) // _BD)

    src3 = _split_edges(edge_index[0], n, k0, k1, _B)
    dst3 = _split_edges(edge_index[1], n, k0, k1, _B)
    dst3d = _pad_edges(edge_index[1], n, _NW, kd, _BD)

    xpad = jnp.zeros((npad, d), x.dtype).at[:n].set(x)
    ones_feat = jnp.ones((npad, d), jnp.float32)
    zfeat = jnp.zeros((npad, d), jnp.float32)
    b1r = b1.reshape(1, d)
    b2r = b2.reshape(1, d)

    degs = _sc_degree(dst3d, ones_feat, zfeat, npad, d, kd)
    g1, dinvs = _tc_g1(xpad, W1, degs, npad, d, bm)
    S1 = _sc_scatter(g1, src3, dst3, zfeat, npad, d, k0, k1)
    pen_pad, g2 = _tc_mid(S1, dinvs, b1r, W2, npad, d, bm)
    S2 = _sc_scatter(g2, src3, dst3, zfeat, npad, d, k0, k1)
    out_pad = _tc_out(S2, dinvs, b2r, npad, d, bm)
    return (out_pad[:n], pen_pad[:n])


# speedup vs baseline: 1.0185x; 1.0185x over previous
import functools

import jax
import jax.numpy as jnp
from jax import lax
from jax.experimental import pallas as pl
from jax.experimental.pallas import tpu as pltpu
from jax.experimental.pallas import tpu_sc as plsc

_NC = 2
_NS = 16
_NW = _NC * _NS
_B = 128
_BD = 128
_KSPLIT = (75, 82)
_MESH = dict(core_axis_name="c", subcore_axis_name="s")


def _sc_scatter(g, src3, dst3, zfeat, npad, d, k0, k1):
    rpt = npad // _NS
    kmax = max(k0, k1)
    mesh = plsc.VectorSubcoreMesh(**_MESH)

    @functools.partial(
        pl.kernel,
        out_type=jax.ShapeDtypeStruct((_NC, npad, d), jnp.float32),
        mesh=mesh,
        scratch_types=[
            pltpu.VMEM((kmax, _B), jnp.int32),
            pltpu.VMEM((kmax, _B), jnp.int32),
            pltpu.VMEM((_B, d), jnp.float32),
            pltpu.VMEM_SHARED((npad, d), jnp.float32),
            pltpu.SemaphoreType.DMA,
        ],
    )
    def scat_kernel(g_hbm, src_hbm, dst_hbm, z_hbm, out_hbm,
                    idx_s, idx_d, rows, acc, sem):
        cid = lax.axis_index("c")
        sid = lax.axis_index("s")
        wid = sid * _NC + cid
        sl = pl.ds(sid * rpt, rpt)

        @pl.when(cid == 0)
        def _():
            pltpu.sync_copy(g_hbm.at[sl], acc.at[sl])

        @pl.when(cid != 0)
        def _():
            pltpu.sync_copy(z_hbm.at[sl], acc.at[sl])

        pltpu.sync_copy(src_hbm.at[wid], idx_s)
        pltpu.sync_copy(dst_hbm.at[wid], idx_d)
        plsc.subcore_barrier()

        kw = jnp.where(cid == 0, k0, k1)

        @pl.loop(0, kw)
        def _(j):
            pltpu.async_copy(g_hbm.at[idx_s.at[j]], rows, sem).wait()
            pltpu.sync_copy(rows, acc.at[idx_d.at[j]], add=True)

        plsc.subcore_barrier()
        pltpu.sync_copy(acc.at[sl], out_hbm.at[cid, sl])

    return scat_kernel(g, src3, dst3, zfeat)


def _sc_degree(dst3, ones_feat, zfeat, npad, d, kd):
    rpt = npad // _NS
    nsem = 4
    mesh = plsc.VectorSubcoreMesh(**_MESH)

    @functools.partial(
        pl.kernel,
        out_type=jax.ShapeDtypeStruct((_NC, npad, d), jnp.float32),
        mesh=mesh,
        scratch_types=[
            pltpu.VMEM((kd, _BD), jnp.int32),
            pltpu.VMEM((_BD, d), jnp.float32),
            pltpu.VMEM_SHARED((npad, d), jnp.float32),
        ] + [pltpu.SemaphoreType.DMA] * nsem,
    )
    def deg_kernel(dst_hbm, ones_hbm, z_hbm, out_hbm, idx_d, ones_v, acc,
                   *sems):
        cid = lax.axis_index("c")
        sid = lax.axis_index("s")
        wid = sid * _NC + cid
        sl = pl.ds(sid * rpt, rpt)

        @pl.when(cid == 0)
        def _():
            pltpu.sync_copy(ones_hbm.at[sl], acc.at[sl])

        @pl.when(cid != 0)
        def _():
            pltpu.sync_copy(z_hbm.at[sl], acc.at[sl])

        pltpu.sync_copy(ones_hbm.at[pl.ds(0, _BD)], ones_v)
        pltpu.sync_copy(dst_hbm.at[wid], idx_d)
        plsc.subcore_barrier()

        @pl.loop(0, kd)
        def _(j):
            for s in range(nsem):
                @pl.when(j % nsem == s)
                def _():
                    pltpu.async_copy(ones_v, acc.at[idx_d.at[j]], sems[s],
                                     add=True)

            @pl.when(j >= nsem)
            def _():
                for s in range(nsem):
                    @pl.when(j % nsem == s)
                    def _():
                        pltpu.make_async_copy(
                            ones_v, acc.at[idx_d.at[j - nsem]],
                            sems[s]).wait()

        @pl.loop(kd - nsem, kd)
        def _(j):
            for s in range(nsem):
                @pl.when(j % nsem == s)
                def _():
                    pltpu.make_async_copy(ones_v, acc.at[idx_d.at[j]],
                                          sems[s]).wait()

        plsc.subcore_barrier()
        pltpu.sync_copy(acc.at[sl], out_hbm.at[cid, sl])

    return deg_kernel(dst3, ones_feat, zfeat)


def _dinv_of(degs_blk):
    deg = degs_blk[0, :, 0] + degs_blk[1, :, 0]
    return lax.rsqrt(deg)


def _tc_g1(xpad, W1, degs, npad, d, bm):
    def body(x_ref, w_ref, degs_ref, g_ref):
        h = jnp.dot(x_ref[...], w_ref[...], preferred_element_type=jnp.float32)
        g_ref[...] = h * _dinv_of(degs_ref)[:, None]

    return pl.pallas_call(
        body,
        grid=(npad // bm,),
        in_specs=[
            pl.BlockSpec((bm, d), lambda i: (i, 0)),
            pl.BlockSpec((d, d), lambda i: (0, 0)),
            pl.BlockSpec((_NC, bm, d), lambda i: (0, i, 0)),
        ],
        out_specs=pl.BlockSpec((bm, d), lambda i: (i, 0)),
        out_shape=jax.ShapeDtypeStruct((npad, d), jnp.float32),
    )(xpad, W1, degs)


def _tc_mid(S1, degs, b1, W2, npad, d, bm):
    def body(s_ref, degs_ref, b_ref, w_ref, pen_ref, g2_ref):
        dinv = _dinv_of(degs_ref)[:, None]
        ssum = s_ref[0] + s_ref[1]
        pen = jnp.maximum(ssum * dinv + b_ref[...], 0.0)
        pen_ref[...] = pen
        h2 = jnp.dot(pen, w_ref[...], preferred_element_type=jnp.float32)
        g2_ref[...] = h2 * dinv

    return pl.pallas_call(
        body,
        grid=(npad // bm,),
        in_specs=[
            pl.BlockSpec((_NC, bm, d), lambda i: (0, i, 0)),
            pl.BlockSpec((_NC, bm, d), lambda i: (0, i, 0)),
            pl.BlockSpec((1, d), lambda i: (0, 0)),
            pl.BlockSpec((d, d), lambda i: (0, 0)),
        ],
        out_specs=[
            pl.BlockSpec((bm, d), lambda i: (i, 0)),
            pl.BlockSpec((bm, d), lambda i: (i, 0)),
        ],
        out_shape=[
            jax.ShapeDtypeStruct((npad, d), jnp.float32),
            jax.ShapeDtypeStruct((npad, d), jnp.float32),
        ],
    )(S1, degs, b1, W2)


def _tc_out(S2, degs, b2, npad, d, bm):
    def body(s_ref, degs_ref, b_ref, out_ref):
        dinv = _dinv_of(degs_ref)[:, None]
        out_ref[...] = (s_ref[0] + s_ref[1]) * dinv + b_ref[...]

    return pl.pallas_call(
        body,
        grid=(npad // bm,),
        in_specs=[
            pl.BlockSpec((_NC, bm, d), lambda i: (0, i, 0)),
            pl.BlockSpec((_NC, bm, d), lambda i: (0, i, 0)),
            pl.BlockSpec((1, d), lambda i: (0, 0)),
        ],
        out_specs=pl.BlockSpec((bm, d), lambda i: (i, 0)),
        out_shape=jax.ShapeDtypeStruct((npad, d), jnp.float32),
    )(S2, degs, b2)


def _pad_edges(idx, padval, nw, k, b):
    epad = nw * k * b
    pad = jnp.full((epad - idx.shape[0],), padval, jnp.int32)
    return jnp.concatenate([idx, pad]).reshape(nw, k, b)


def _split_edges(idx, padval, k0, k1, b):
    na = _NS * k0 * b
    nb = _NS * k1 * b
    pad = jnp.full((na + nb - idx.shape[0],), padval, jnp.int32)
    full = jnp.concatenate([idx, pad])
    part_a = full[:na].reshape(_NS, k0, b)
    part_b = full[na:].reshape(_NS, k1, b)
    kmax = max(k0, k1)

    def widen(p, kk):
        if kk == kmax:
            return p
        fill = jnp.full((_NS, kmax - kk, b), padval, jnp.int32)
        return jnp.concatenate([p, fill], axis=1)

    return jnp.stack([widen(part_a, k0), widen(part_b, k1)],
                     axis=1).reshape(_NW, kmax, b)


def kernel(x, edge_index, W1, b1, W2, b2):
    n, d = x.shape
    e = edge_index.shape[1]
    bm = 256
    npad = -(-n // bm) * bm
    k0, k1 = _KSPLIT
    kd = -(-(-(-e // _NW)) // _BD)

    src3 = _split_edges(edge_index[0], n, k0, k1, _B)
    dst3 = _split_edges(edge_index[1], n, k0, k1, _B)
    dst3d = _pad_edges(edge_index[1], n, _NW, kd, _BD)

    xpad = jnp.zeros((npad, d), x.dtype).at[:n].set(x)
    ones_feat = jnp.ones((npad, d), jnp.float32)
    zfeat = jnp.zeros((npad, d), jnp.float32)
    b1r = b1.reshape(1, d)
    b2r = b2.reshape(1, d)

    degs = _sc_degree(dst3d, ones_feat, zfeat, npad, d, kd)
    g1 = _tc_g1(xpad, W1, degs, npad, d, bm)
    S1 = _sc_scatter(g1, src3, dst3, zfeat, npad, d, k0, k1)
    pen_pad, g2 = _tc_mid(S1, degs, b1r, W2, npad, d, bm)
    S2 = _sc_scatter(g2, src3, dst3, zfeat, npad, d, k0, k1)
    out_pad = _tc_out(S2, degs, b2r, npad, d, bm)
    return (out_pad[:n], pen_pad[:n])
